# baseline (speedup 1.0000x reference)
import jax
import jax.numpy as jnp
from jax.experimental import pallas as pl
from jax.experimental.pallas import tpu as pltpu


def _make_body(A, G):
    def body(act_ref, y_ref, w_ref, binit_ref, x1_ref, x2_ref, mask_ref,
             out_ref, wy_ref):
        i = pl.program_id(0)
        B = act_ref.shape[0]

        @pl.when(i == 0)
        def _init():
            wy_ref[...] = binit_ref[...].reshape(G, 4, -1)

        @pl.when(i < A)
        def _accumulate():
            sel = jnp.where(act_ref[...] == i, y_ref[...], 0.0)
            wy_ref[...] += jax.lax.dot_general(
                sel, w_ref[0], (((1,), (0,)), ((), ())),
                preferred_element_type=jnp.float32).reshape(G, 4, -1)

        @pl.when(i >= A)
        def _scores():
            j = i - A
            wyblk = wy_ref[j]
            halves = []
            for xr in (x1_ref, x2_ref):
                rows = []
                for k in range(4):
                    rows.append(jax.lax.dot_general(
                        wyblk[k:k + 1, :], xr[0, k, 0],
                        (((1,), (1,)), ((), ())),
                        preferred_element_type=jnp.float32))
                halves.append(jnp.concatenate(rows, axis=0))
            out_ref[j] = jnp.concatenate(halves, axis=1)

        @pl.when(i == A + G - 1)
        def _epilogue():
            s = out_ref[...]
            s = jnp.where(mask_ref[...] != 0, -jnp.inf, s)
            m = jnp.max(s, axis=-1, keepdims=True)
            sh = s - m
            lse = jnp.log(jnp.sum(jnp.exp(sh), axis=-1, keepdims=True))
            out_ref[...] = sh - lse

    return body


def kernel(x, y, x_mask, actions, weight, bias):
    B, L, X = x.shape
    A, Y, _ = weight.shape
    G = B // 4
    actions = actions.astype(jnp.int32)
    act2d = actions.reshape(B, 1)
    bias_g = jnp.take(bias, actions, axis=0)
    x5 = x.reshape(G, 4, 2, L // 2, X)
    mask4 = x_mask.astype(jnp.int32).reshape(G, 4, L)

    out = pl.pallas_call(
        _make_body(A, G),
        grid=(A + G,),
        in_specs=[
            pl.BlockSpec((B, 1), lambda i: (0, 0)),
            pl.BlockSpec((B, Y), lambda i: (0, 0)),
            pl.BlockSpec((1, Y, X), lambda i: (jnp.minimum(i, A - 1), 0, 0)),
            pl.BlockSpec((B, X), lambda i: (0, 0)),
            pl.BlockSpec((1, 4, 1, L // 2, X),
                         lambda i: (jnp.maximum(i - A, 0), 0, 0, 0, 0)),
            pl.BlockSpec((1, 4, 1, L // 2, X),
                         lambda i: (jnp.maximum(i - A, 0), 0, 1, 0, 0)),
            pl.BlockSpec((G, 4, L), lambda i: (0, 0, 0)),
        ],
        out_specs=pl.BlockSpec((G, 4, L), lambda i: (0, 0, 0)),
        out_shape=jax.ShapeDtypeStruct((G, 4, L), jnp.float32),
        scratch_shapes=[pltpu.VMEM((G, 4, X), jnp.float32)],
        compiler_params=pltpu.CompilerParams(
            dimension_semantics=("arbitrary",),
            vmem_limit_bytes=52 * 1024 * 1024,
        ),
        name="bilinear_seq_attn_fused",
    )(act2d, y, weight, bias_g, x5, x5, mask4)
    return out.reshape(B, L)


# speedup vs baseline: 1.0092x; 1.0092x over previous
import jax
import jax.numpy as jnp
from jax.experimental import pallas as pl
from jax.experimental.pallas import tpu as pltpu


def _make_body(A, G):
    def body(act_ref, y_ref, w_ref, b_ref, x1_ref, x2_ref, mask_ref,
             out_ref, wy_ref):
        i = pl.program_id(0)

        @pl.when(i == 0)
        def _init():
            wy_ref[...] = jnp.zeros_like(wy_ref)

        @pl.when(i < A)
        def _accumulate():
            hit = act_ref[...] == i
            sel = jnp.where(hit, y_ref[...], 0.0)
            contrib = jax.lax.dot_general(
                sel, w_ref[0], (((1,), (0,)), ((), ())),
                preferred_element_type=jnp.float32)
            contrib += jnp.where(hit, b_ref[i], 0.0)
            wy_ref[...] += contrib.reshape(G, 4, -1)

        def _softmax_block(j):
            s = out_ref[j]
            s = jnp.where(mask_ref[j] != 0, -jnp.inf, s)
            m = jnp.max(s, axis=-1, keepdims=True)
            sh = s - m
            lse = jnp.log(jnp.sum(jnp.exp(sh), axis=-1, keepdims=True))
            out_ref[j] = sh - lse

        @pl.when(i >= A)
        def _scores():
            j = i - A
            wyblk = wy_ref[j]
            halves = []
            for xr in (x1_ref, x2_ref):
                rows = []
                for k in range(4):
                    rows.append(jax.lax.dot_general(
                        wyblk[k:k + 1, :], xr[0, k, 0],
                        (((1,), (1,)), ((), ())),
                        preferred_element_type=jnp.float32))
                halves.append(jnp.concatenate(rows, axis=0))
            out_ref[j] = jnp.concatenate(halves, axis=1)

            @pl.when(j >= 1)
            def _():
                _softmax_block(j - 1)

            @pl.when(j == G - 1)
            def _():
                _softmax_block(G - 1)

    return body


def kernel(x, y, x_mask, actions, weight, bias):
    B, L, X = x.shape
    A, Y, _ = weight.shape
    G = B // 4
    actions = actions.astype(jnp.int32)
    act2d = actions.reshape(B, 1)
    bias3 = bias.reshape(A, 1, X)
    x5 = x.reshape(G, 4, 2, L // 2, X)
    mask4 = x_mask.astype(jnp.int32).reshape(G, 4, L)

    out = pl.pallas_call(
        _make_body(A, G),
        grid=(A + G,),
        in_specs=[
            pl.BlockSpec((B, 1), lambda i: (0, 0)),
            pl.BlockSpec((B, Y), lambda i: (0, 0)),
            pl.BlockSpec((1, Y, X), lambda i: (jnp.minimum(i, A - 1), 0, 0)),
            pl.BlockSpec((A, 1, X), lambda i: (0, 0, 0)),
            pl.BlockSpec((1, 4, 1, L // 2, X),
                         lambda i: (jnp.maximum(i - A, 0), 0, 0, 0, 0)),
            pl.BlockSpec((1, 4, 1, L // 2, X),
                         lambda i: (jnp.maximum(i - A, 0), 0, 1, 0, 0)),
            pl.BlockSpec((G, 4, L), lambda i: (0, 0, 0)),
        ],
        out_specs=pl.BlockSpec((G, 4, L), lambda i: (0, 0, 0)),
        out_shape=jax.ShapeDtypeStruct((G, 4, L), jnp.float32),
        scratch_shapes=[pltpu.VMEM((G, 4, X), jnp.float32)],
        compiler_params=pltpu.CompilerParams(
            dimension_semantics=("arbitrary",),
            vmem_limit_bytes=52 * 1024 * 1024,
        ),
        name="bilinear_seq_attn_fused",
    )(act2d, y, weight, bias3, x5, x5, mask4)
    return out.reshape(B, L)
